# baseline (speedup 1.0000x reference)
import functools

import jax
import jax.numpy as jnp
from jax import lax
from jax.experimental import pallas as pl
from jax.experimental.pallas import tpu as pltpu
from jax.experimental.pallas import tpu_sc as plsc

D = 128
CB = 128
NW = 32
N_SUB = 16


G = 8


def _sc_aggregate(h, zeros_pad, src_t, dst_t, ch, npad):
    ngroups = ch // G
    rows_per_tile = npad // N_SUB
    mesh = plsc.VectorSubcoreMesh(core_axis_name="c", subcore_axis_name="s")

    @functools.partial(
        pl.kernel,
        out_type=jax.ShapeDtypeStruct((2, npad, D), jnp.float32),
        mesh=mesh,
        scratch_types=[
            pltpu.VMEM((2, G, CB), jnp.int32),
            pltpu.VMEM((2, G, CB), jnp.int32),
            pltpu.VMEM((CB, D), jnp.float32),
            pltpu.VMEM_SHARED((npad, D), jnp.float32),
            pltpu.SemaphoreType.DMA,
            pltpu.SemaphoreType.DMA,
            pltpu.SemaphoreType.DMA,
        ],
    )
    def agg(h_hbm, z_hbm, src_hbm, dst_hbm, out_hbm,
            src_v, dst_v, rows_v, acc, gsem, ssem, isem):
        cid = lax.axis_index("c")
        sid = lax.axis_index("s")
        wid = cid * N_SUB + sid
        r0 = sid * rows_per_tile
        pltpu.sync_copy(z_hbm.at[pl.ds(r0, rows_per_tile)],
                        acc.at[pl.ds(r0, rows_per_tile)])
        pltpu.sync_copy(src_hbm.at[wid, 0], src_v.at[0])
        pltpu.sync_copy(dst_hbm.at[wid, 0], dst_v.at[0])
        plsc.subcore_barrier()

        def gather(ip, c):
            return pltpu.make_async_copy(
                h_hbm.at[src_v.at[ip, c]], rows_v, gsem)

        def scatter(ip, c):
            return pltpu.make_async_copy(
                rows_v, acc.at[dst_v.at[ip, c]], ssem)

        def idx_load(g, ip):
            return (pltpu.make_async_copy(src_hbm.at[wid, g], src_v.at[ip],
                                          isem),
                    pltpu.make_async_copy(dst_hbm.at[wid, g], dst_v.at[ip],
                                          isem))

        def group(g, ip):
            @pl.when(g + 1 < ngroups)
            def _():
                for cp in idx_load(g + 1, 1 - ip):
                    cp.start()

            for c in range(G):
                cp = gather(ip, c)
                cp.start()
                cp.wait()
                sc = scatter(ip, c)
                sc.start(add=True)
                sc.wait()

            @pl.when(g + 1 < ngroups)
            def _():
                for cp in idx_load(g + 1, 1 - ip):
                    cp.wait()

        def body(i, carry):
            group(2 * i, 0)
            group(2 * i + 1, 1)
            return carry

        lax.fori_loop(0, ngroups // 2, body, 0)
        plsc.subcore_barrier()
        pltpu.sync_copy(acc.at[pl.ds(r0, rows_per_tile)],
                        out_hbm.at[cid, pl.ds(r0, rows_per_tile)])

    return agg(h, zeros_pad, src_t, dst_t)


def _mlp_call(partials, h, Wa, ba, Wb, bb, final_relu):
    n = h.shape[0]
    br = 1000
    grid = (n // br,)

    def body(p_ref, h_ref, wa_ref, ba_ref, wb_ref, bb_ref, o_ref):
        a = h_ref[...] + p_ref[0] + p_ref[1]
        t = jnp.dot(a, wa_ref[...], preferred_element_type=jnp.float32)
        t = jnp.maximum(t + ba_ref[...], 0.0)
        t = jnp.dot(t, wb_ref[...], preferred_element_type=jnp.float32)
        t = t + bb_ref[...]
        if final_relu:
            t = jnp.maximum(t, 0.0)
        o_ref[...] = t

    return pl.pallas_call(
        body,
        grid=grid,
        in_specs=[
            pl.BlockSpec((2, br, D), lambda i: (0, i, 0)),
            pl.BlockSpec((br, D), lambda i: (i, 0)),
            pl.BlockSpec((D, D), lambda i: (0, 0)),
            pl.BlockSpec((1, D), lambda i: (0, 0)),
            pl.BlockSpec((D, D), lambda i: (0, 0)),
            pl.BlockSpec((1, D), lambda i: (0, 0)),
        ],
        out_specs=pl.BlockSpec((br, D), lambda i: (i, 0)),
        out_shape=jax.ShapeDtypeStruct((n, D), jnp.float32),
    )(partials, h, Wa, ba.reshape(1, D), Wb, bb.reshape(1, D))


def kernel(x, edge_index, W1a, b1a, W1b, b1b, W2a, b2a, W2b, b2b):
    n = x.shape[0]
    npad = ((n + 127) // 128) * 128 + 128 if n % 128 == 0 else -(-n // 128) * 128
    src = edge_index[0].astype(jnp.int32)
    dst = edge_index[1].astype(jnp.int32)
    e = src.shape[0]
    per_tile = -(-e // NW)
    ch = -(-per_tile // CB)
    ch = -(-ch // (2 * G)) * (2 * G)
    e_pad = NW * ch * CB
    src_p = jnp.concatenate(
        [src, jnp.zeros((e_pad - e,), jnp.int32)]).reshape(NW, ch // G, G, CB)
    dst_p = jnp.concatenate(
        [dst, jnp.full((e_pad - e,), n, jnp.int32)]).reshape(NW, ch // G, G, CB)
    zeros_pad = jnp.zeros((npad, D), jnp.float32)

    p1 = _sc_aggregate(x, zeros_pad, src_p, dst_p, ch, npad)
    h1 = _mlp_call(p1, x, W1a, b1a, W1b, b1b, final_relu=True)
    p2 = _sc_aggregate(h1, zeros_pad, src_p, dst_p, ch, npad)
    out = _mlp_call(p2, h1, W2a, b2a, W2b, b2b, final_relu=False)
    return out


# speedup vs baseline: 1.5073x; 1.5073x over previous
import functools

import jax
import jax.numpy as jnp
from jax import lax
from jax.experimental import pallas as pl
from jax.experimental.pallas import tpu as pltpu
from jax.experimental.pallas import tpu_sc as plsc

D = 128
CB = 128
NW = 32
N_SUB = 16


def _sc_aggregate(h, zeros_pad, src_t, dst_t, ch, npad):
    rows_per_tile = npad // N_SUB
    mesh = plsc.VectorSubcoreMesh(core_axis_name="c", subcore_axis_name="s")

    @functools.partial(
        pl.kernel,
        out_type=jax.ShapeDtypeStruct((2, npad, D), jnp.float32),
        mesh=mesh,
        scratch_types=[
            pltpu.VMEM((ch, CB), jnp.int32),
            pltpu.VMEM((ch, CB), jnp.int32),
            pltpu.VMEM((CB, D), jnp.float32),
            pltpu.VMEM_SHARED((npad, D), jnp.float32),
            pltpu.SemaphoreType.DMA,
        ],
    )
    def agg(h_hbm, z_hbm, src_hbm, dst_hbm, out_hbm,
            src_v, dst_v, rows_v, acc, sem):
        cid = lax.axis_index("c")
        sid = lax.axis_index("s")
        wid = cid * N_SUB + sid
        r0 = sid * rows_per_tile
        pltpu.sync_copy(z_hbm.at[pl.ds(r0, rows_per_tile)],
                        acc.at[pl.ds(r0, rows_per_tile)])
        pltpu.sync_copy(src_hbm.at[wid], src_v)
        pltpu.sync_copy(dst_hbm.at[wid], dst_v)
        plsc.subcore_barrier()

        def body(j, carry):
            pltpu.async_copy(h_hbm.at[src_v.at[j]], rows_v, sem).wait()
            pltpu.sync_copy(rows_v, acc.at[dst_v.at[j]], add=True)
            return carry

        lax.fori_loop(0, ch, body, 0)
        plsc.subcore_barrier()
        pltpu.sync_copy(acc.at[pl.ds(r0, rows_per_tile)],
                        out_hbm.at[cid, pl.ds(r0, rows_per_tile)])

    return agg(h, zeros_pad, src_t, dst_t)


def _mlp_call(partials, h, Wa, ba, Wb, bb, final_relu):
    n = h.shape[0]
    br = 1000
    grid = (n // br,)

    def body(p_ref, h_ref, wa_ref, ba_ref, wb_ref, bb_ref, o_ref):
        a = h_ref[...] + p_ref[0] + p_ref[1]
        t = jnp.dot(a, wa_ref[...], preferred_element_type=jnp.float32)
        t = jnp.maximum(t + ba_ref[...], 0.0)
        t = jnp.dot(t, wb_ref[...], preferred_element_type=jnp.float32)
        t = t + bb_ref[...]
        if final_relu:
            t = jnp.maximum(t, 0.0)
        o_ref[...] = t

    return pl.pallas_call(
        body,
        grid=grid,
        in_specs=[
            pl.BlockSpec((2, br, D), lambda i: (0, i, 0)),
            pl.BlockSpec((br, D), lambda i: (i, 0)),
            pl.BlockSpec((D, D), lambda i: (0, 0)),
            pl.BlockSpec((1, D), lambda i: (0, 0)),
            pl.BlockSpec((D, D), lambda i: (0, 0)),
            pl.BlockSpec((1, D), lambda i: (0, 0)),
        ],
        out_specs=pl.BlockSpec((br, D), lambda i: (i, 0)),
        out_shape=jax.ShapeDtypeStruct((n, D), jnp.float32),
    )(partials, h, Wa, ba.reshape(1, D), Wb, bb.reshape(1, D))


def kernel(x, edge_index, W1a, b1a, W1b, b1b, W2a, b2a, W2b, b2b):
    n = x.shape[0]
    npad = ((n + 127) // 128) * 128 + 128 if n % 128 == 0 else -(-n // 128) * 128
    src = edge_index[0].astype(jnp.int32)
    dst = edge_index[1].astype(jnp.int32)
    e = src.shape[0]
    per_tile = -(-e // NW)
    ch = -(-per_tile // CB)
    e_pad = NW * ch * CB
    src_p = jnp.concatenate(
        [src, jnp.zeros((e_pad - e,), jnp.int32)]).reshape(NW, ch, CB)
    dst_p = jnp.concatenate(
        [dst, jnp.full((e_pad - e,), n, jnp.int32)]).reshape(NW, ch, CB)
    zeros_pad = jnp.zeros((npad, D), jnp.float32)

    p1 = _sc_aggregate(x, zeros_pad, src_p, dst_p, ch, npad)
    h1 = _mlp_call(p1, x, W1a, b1a, W1b, b1b, final_relu=True)
    p2 = _sc_aggregate(h1, zeros_pad, src_p, dst_p, ch, npad)
    out = _mlp_call(p2, h1, W2a, b2a, W2b, b2b, final_relu=False)
    return out


# speedup vs baseline: 2.0567x; 1.3645x over previous
import functools

import jax
import jax.numpy as jnp
from jax import lax
from jax.experimental import pallas as pl
from jax.experimental.pallas import tpu as pltpu
from jax.experimental.pallas import tpu_sc as plsc

D = 128
CB = 128
NW = 32
N_SUB = 16


def _sc_aggregate(h, zeros_pad, src_t, dst_t, ch0, ch1, npad):
    chmax = max(ch0, ch1)
    rows_per_tile = npad // N_SUB
    mesh = plsc.VectorSubcoreMesh(core_axis_name="c", subcore_axis_name="s")

    @functools.partial(
        pl.kernel,
        out_type=jax.ShapeDtypeStruct((2, npad, D), jnp.float32),
        mesh=mesh,
        scratch_types=[
            pltpu.VMEM((chmax, CB), jnp.int32),
            pltpu.VMEM((chmax, CB), jnp.int32),
            pltpu.VMEM((CB, D), jnp.float32),
            pltpu.VMEM_SHARED((npad, D), jnp.float32),
            pltpu.SemaphoreType.DMA,
        ],
    )
    def agg(h_hbm, z_hbm, src_hbm, dst_hbm, out_hbm,
            src_v, dst_v, rows_v, acc, sem):
        cid = lax.axis_index("c")
        sid = lax.axis_index("s")
        wid = cid * N_SUB + sid
        r0 = sid * rows_per_tile
        pltpu.sync_copy(z_hbm.at[pl.ds(r0, rows_per_tile)],
                        acc.at[pl.ds(r0, rows_per_tile)])
        pltpu.sync_copy(src_hbm.at[wid], src_v)
        pltpu.sync_copy(dst_hbm.at[wid], dst_v)
        plsc.subcore_barrier()

        def body(j, carry):
            pltpu.async_copy(h_hbm.at[src_v.at[j]], rows_v, sem).wait()
            pltpu.sync_copy(rows_v, acc.at[dst_v.at[j]], add=True)
            return carry

        my_ch = jnp.where(cid == 0, ch0, ch1)
        lax.fori_loop(0, my_ch, body, 0)
        plsc.subcore_barrier()
        pltpu.sync_copy(acc.at[pl.ds(r0, rows_per_tile)],
                        out_hbm.at[cid, pl.ds(r0, rows_per_tile)])

    return agg(h, zeros_pad, src_t, dst_t)


def _mlp_call(partials, h, Wa, ba, Wb, bb, final_relu):
    n = h.shape[0]
    br = 1000
    grid = (n // br,)

    def body(p_ref, h_ref, wa_ref, ba_ref, wb_ref, bb_ref, o_ref):
        a = h_ref[...] + p_ref[0] + p_ref[1]
        t = jnp.dot(a, wa_ref[...], preferred_element_type=jnp.float32)
        t = jnp.maximum(t + ba_ref[...], 0.0)
        t = jnp.dot(t, wb_ref[...], preferred_element_type=jnp.float32)
        t = t + bb_ref[...]
        if final_relu:
            t = jnp.maximum(t, 0.0)
        o_ref[...] = t

    return pl.pallas_call(
        body,
        grid=grid,
        in_specs=[
            pl.BlockSpec((2, br, D), lambda i: (0, i, 0)),
            pl.BlockSpec((br, D), lambda i: (i, 0)),
            pl.BlockSpec((D, D), lambda i: (0, 0)),
            pl.BlockSpec((1, D), lambda i: (0, 0)),
            pl.BlockSpec((D, D), lambda i: (0, 0)),
            pl.BlockSpec((1, D), lambda i: (0, 0)),
        ],
        out_specs=pl.BlockSpec((br, D), lambda i: (i, 0)),
        out_shape=jax.ShapeDtypeStruct((n, D), jnp.float32),
    )(partials, h, Wa, ba.reshape(1, D), Wb, bb.reshape(1, D))


def kernel(x, edge_index, W1a, b1a, W1b, b1b, W2a, b2a, W2b, b2b):
    n = x.shape[0]
    npad = ((n + 127) // 128) * 128 + 128 if n % 128 == 0 else -(-n // 128) * 128
    src = edge_index[0].astype(jnp.int32)
    dst = edge_index[1].astype(jnp.int32)
    e = src.shape[0]
    f_fast = 0.63
    chunks = -(-e // CB)
    ch0 = max(1, min(int(round(f_fast * chunks / N_SUB)), chunks // N_SUB))
    ch1 = max(1, -(-(chunks - N_SUB * ch0) // N_SUB))
    chmax = max(ch0, ch1)
    cap0 = N_SUB * ch0 * CB
    cap1 = N_SUB * ch1 * CB
    src_f = jnp.concatenate([src, jnp.zeros((cap0 + cap1 - e,), jnp.int32)])
    dst_f = jnp.concatenate([dst, jnp.full((cap0 + cap1 - e,), n, jnp.int32)])

    def per_tile(flat, fill):
        a0 = flat[:cap0].reshape(N_SUB, ch0, CB)
        a1 = flat[cap0:].reshape(N_SUB, ch1, CB)
        a0 = jnp.pad(a0, ((0, 0), (0, chmax - ch0), (0, 0)),
                     constant_values=fill)
        a1 = jnp.pad(a1, ((0, 0), (0, chmax - ch1), (0, 0)),
                     constant_values=fill)
        return jnp.concatenate([a0, a1])

    src_p = per_tile(src_f, 0)
    dst_p = per_tile(dst_f, n)
    zeros_pad = jnp.zeros((npad, D), jnp.float32)

    p1 = _sc_aggregate(x, zeros_pad, src_p, dst_p, ch0, ch1, npad)
    h1 = _mlp_call(p1, x, W1a, b1a, W1b, b1b, final_relu=True)
    p2 = _sc_aggregate(h1, zeros_pad, src_p, dst_p, ch0, ch1, npad)
    out = _mlp_call(p2, h1, W2a, b2a, W2b, b2b, final_relu=False)
    return out
